# baseline (speedup 1.0000x reference)
import functools

import jax
import jax.numpy as jnp
from jax import lax
from jax.experimental import pallas as pl
from jax.experimental.pallas import tpu as pltpu
from jax.experimental.pallas import tpu_sc as plsc

_R = 144


def _rownorm_sumsq(v):
    sq = v * v
    n = sq.shape[1]
    s = sq[:, 0:8]
    for k in range(1, n // 8):
        s = s + sq[:, 8 * k:8 * (k + 1)]
    w = 8
    while w > 1:
        s = s[:, :w // 2] + s[:, w // 2:w]
        w //= 2
    return s


def _select_body(m_ref, idx_ref, w_ref, *, half, r, big, nb):
    for bb in range(nb):
        _select_one(m_ref, idx_ref, w_ref, bb, half=half, r=r, big=big)


def _select_one(m_ref, idx_ref, w_ref, bb, *, half, r, big):
    mm = m_ref[bb]
    d = mm.shape[1] // 2
    a = mm[:, :d]
    b = mm[:, d:]
    a = a / jnp.sqrt(_rownorm_sumsq(a))
    b = b / jnp.sqrt(_rownorm_sumsq(b))
    s = lax.dot_general(a, b, (((1,), (1,)), ((), ())),
                        preferred_element_type=jnp.float32)

    nm = jnp.max(s, axis=1, keepdims=True)
    ii = lax.broadcasted_iota(jnp.int32, (half, half), 0)
    jj = lax.broadcasted_iota(jnp.int32, (half, half), 1)
    nidx = jnp.min(jnp.where(s == nm, jj, big), axis=1, keepdims=True)

    nm_row = jnp.swapaxes(nm, 0, 1)

    cmp = ((nm_row > nm) | ((nm_row == nm) & (jj < ii))).astype(jnp.float32)
    rank = lax.dot_general(cmp, jnp.ones((half, 1), jnp.float32),
                           (((1,), (0,)), ((), ())),
                           preferred_element_type=jnp.float32)
    rank = rank.astype(jnp.int32)

    unm = half - r
    o_idx = jnp.where(rank >= r, rank - r, unm + nidx)
    nout = unm + half
    b_global = pl.program_id(0) * 4 + bb
    o_idx = o_idx + (b_global // 8) * nout
    idx_ref[bb] = jnp.swapaxes(o_idx, 0, 1)

    dmat = ((nidx == jj) & (rank < r)).astype(jnp.float32)
    cnt = lax.dot_general(jnp.ones((1, half), jnp.float32), dmat,
                          (((1,), (0,)), ((), ())),
                          preferred_element_type=jnp.float32)
    invw = 1.0 / (1.0 + jnp.swapaxes(cnt, 0, 1))
    w_ref[bb] = jnp.broadcast_to(invw, (half, 16))


_F = 128
_NSTREAM = 3


def _sc_merge_body(x2_hbm, idx_hbm, invw_hbm, zz_hbm, out_hbm,
                   esrc, idxv, wv, acc_sh, *, half, unm, c, bpw, nch):
    nc = 2
    wid = lax.axis_index("s") * nc + lax.axis_index("c")
    nout = unm + half
    a0 = lax.axis_index("s") * nout
    for bb in range(bpw):
        bidx = wid * bpw + bb
        pltpu.sync_copy(idx_hbm.at[bidx], idxv)
        pltpu.sync_copy(invw_hbm.at[bidx], wv)
        for ch in range(nch):
            c0 = ch * _F
            pltpu.sync_copy(zz_hbm, acc_sh.at[pl.ds(a0, unm)])
            pltpu.sync_copy(x2_hbm.at[bidx, :, pl.ds(c + c0, _F)],
                            acc_sh.at[pl.ds(a0 + unm, half)])
            pltpu.sync_copy(x2_hbm.at[bidx, :, pl.ds(c0, _F)], esrc)
            for j in range(_NSTREAM):
                pltpu.sync_copy(esrc.at[pl.ds(j * 96, 96)],
                                acc_sh.at[idxv.at[j]], add=True)

            pltpu.sync_copy(acc_sh.at[pl.ds(a0, unm)],
                            out_hbm.at[bidx, pl.ds(0, unm), pl.ds(c0, _F)])
            pltpu.sync_copy(acc_sh.at[pl.ds(a0 + unm, half)], esrc)

            def mrow(rr, carry):
                w16 = wv[rr]
                for k in range(_F // 16):
                    sl = pl.ds(k * 16, 16)
                    esrc[rr, sl] = esrc[rr, sl] * w16
                return carry

            lax.fori_loop(0, half, mrow, 0)
            pltpu.sync_copy(esrc,
                            out_hbm.at[bidx, pl.ds(unm, half), pl.ds(c0, _F)])


def kernel(x, metric):
    bsz, t, c = x.shape
    d = metric.shape[-1]
    half = t // 2
    r = min(_R, half)
    unm = half - r
    nout = unm + half

    x2 = x.reshape(bsz, half, 2 * c)
    m2 = metric.reshape(bsz, half, 2 * d)

    nb = 4
    body = functools.partial(_select_body, half=half, r=r, big=1 << 30,
                             nb=nb)
    idx_all, invw = pl.pallas_call(
        body,
        grid=(bsz // nb,),
        in_specs=[
            pl.BlockSpec((nb, half, 2 * d), lambda i: (i, 0, 0)),
        ],
        out_specs=[
            pl.BlockSpec((nb, 1, half), lambda i: (i, 0, 0)),
            pl.BlockSpec((nb, half, 16), lambda i: (i, 0, 0)),
        ],
        out_shape=[
            jax.ShapeDtypeStruct((bsz, 1, half), jnp.int32),
            jax.ShapeDtypeStruct((bsz, half, 16), jnp.float32),
        ],
    )(m2)

    idx3 = idx_all.reshape(bsz, _NSTREAM, half // _NSTREAM)
    zz = jnp.zeros((unm, _F), jnp.float32)

    nw = 32
    bpw = bsz // nw
    nch = c // _F
    mesh = plsc.VectorSubcoreMesh(core_axis_name="c", subcore_axis_name="s")
    sc_body = functools.partial(_sc_merge_body, half=half, unm=unm, c=c,
                                bpw=bpw, nch=nch)
    merge = pl.kernel(
        sc_body,
        mesh=mesh,
        out_type=jax.ShapeDtypeStruct((bsz, nout, c), jnp.float32),
        scratch_types=[
            pltpu.VMEM((half, _F), jnp.float32),
            pltpu.VMEM((_NSTREAM, half // _NSTREAM), jnp.int32),
            pltpu.VMEM((half, 16), jnp.float32),
            pltpu.VMEM_SHARED((16 * nout, _F), jnp.float32),
        ],
    )
    return merge(x2, idx3, invw, zz)


# speedup vs baseline: 1.0778x; 1.0778x over previous
import functools

import jax
import jax.numpy as jnp
from jax import lax
from jax.experimental import pallas as pl
from jax.experimental.pallas import tpu as pltpu
from jax.experimental.pallas import tpu_sc as plsc

_R = 144


def _rownorm_sumsq(v):
    sq = v * v
    n = sq.shape[1]
    s = sq[:, 0:8]
    for k in range(1, n // 8):
        s = s + sq[:, 8 * k:8 * (k + 1)]
    w = 8
    while w > 1:
        s = s[:, :w // 2] + s[:, w // 2:w]
        w //= 2
    return s


def _select_body(m_ref, idx_ref, w_ref, *, half, r, big, nb):
    for bb in range(nb):
        _select_one(m_ref, idx_ref, w_ref, bb, half=half, r=r, big=big)


def _select_one(m_ref, idx_ref, w_ref, bb, *, half, r, big):
    mm = m_ref[bb]
    d = mm.shape[1] // 2
    a = mm[:, :d]
    b = mm[:, d:]
    a = a / jnp.sqrt(_rownorm_sumsq(a))
    b = b / jnp.sqrt(_rownorm_sumsq(b))
    s = lax.dot_general(a, b, (((1,), (1,)), ((), ())),
                        preferred_element_type=jnp.float32)

    nm = jnp.max(s, axis=1, keepdims=True)
    ii = lax.broadcasted_iota(jnp.int32, (half, half), 0)
    jj = lax.broadcasted_iota(jnp.int32, (half, half), 1)
    nidx = jnp.min(jnp.where(s == nm, jj, big), axis=1, keepdims=True)

    nm_row = jnp.swapaxes(nm, 0, 1)

    cmp = ((nm_row > nm) | ((nm_row == nm) & (jj < ii))).astype(jnp.float32)
    rank = lax.dot_general(cmp, jnp.ones((half, 1), jnp.float32),
                           (((1,), (0,)), ((), ())),
                           preferred_element_type=jnp.float32)
    rank = rank.astype(jnp.int32)

    unm = half - r
    o_idx = jnp.where(rank >= r, rank - r, unm + nidx)
    nout = unm + half
    b_global = pl.program_id(0) * 4 + bb
    o_idx = o_idx + (b_global // 8) * nout
    idx_ref[bb] = jnp.swapaxes(o_idx, 0, 1)

    dmat = ((nidx == jj) & (rank < r)).astype(jnp.float32)
    cnt = lax.dot_general(jnp.ones((1, half), jnp.float32), dmat,
                          (((1,), (0,)), ((), ())),
                          preferred_element_type=jnp.float32)
    invw = 1.0 / (1.0 + jnp.swapaxes(cnt, 0, 1))
    w_ref[bb] = jnp.broadcast_to(invw, (half, 16))


_F = 128
_NSTREAM = 3


def _sc_merge_body(x2_hbm, idx_hbm, invw_hbm, zz_hbm, out_hbm,
                   esrc, idxv, wv, sem0, sem1, sem2, acc_sh,
                   *, half, unm, c, bpw, nch):
    nc = 2
    wid = lax.axis_index("s") * nc + lax.axis_index("c")
    nout = unm + half
    a0 = lax.axis_index("s") * nout
    for bb in range(bpw):
        bidx = wid * bpw + bb
        pltpu.sync_copy(idx_hbm.at[bidx], idxv)
        pltpu.sync_copy(invw_hbm.at[bidx], wv)
        for ch in range(nch):
            c0 = ch * _F
            h0 = pltpu.async_copy(zz_hbm, acc_sh.at[pl.ds(a0, unm)], sem0)
            h1 = pltpu.async_copy(x2_hbm.at[bidx, :, pl.ds(c + c0, _F)],
                                  acc_sh.at[pl.ds(a0 + unm, half)], sem1)
            h2 = pltpu.async_copy(x2_hbm.at[bidx, :, pl.ds(c0, _F)],
                                  esrc, sem2)
            h0.wait()
            h1.wait()
            h2.wait()
            for j in range(_NSTREAM):
                pltpu.sync_copy(esrc.at[pl.ds(j * 96, 96)],
                                acc_sh.at[idxv.at[j]], add=True)

            h0 = pltpu.async_copy(acc_sh.at[pl.ds(a0, unm)],
                                  out_hbm.at[bidx, pl.ds(0, unm),
                                             pl.ds(c0, _F)], sem0)
            pltpu.sync_copy(acc_sh.at[pl.ds(a0 + unm, half)], esrc)

            def mrow(rr, carry):
                w16 = wv[rr]
                for k in range(_F // 16):
                    sl = pl.ds(k * 16, 16)
                    esrc[rr, sl] = esrc[rr, sl] * w16
                return carry

            lax.fori_loop(0, half, mrow, 0)
            h0.wait()
            pltpu.sync_copy(esrc,
                            out_hbm.at[bidx, pl.ds(unm, half), pl.ds(c0, _F)])


def kernel(x, metric):
    bsz, t, c = x.shape
    d = metric.shape[-1]
    half = t // 2
    r = min(_R, half)
    unm = half - r
    nout = unm + half

    x2 = x.reshape(bsz, half, 2 * c)
    m2 = metric.reshape(bsz, half, 2 * d)

    nb = 4
    body = functools.partial(_select_body, half=half, r=r, big=1 << 30,
                             nb=nb)
    idx_all, invw = pl.pallas_call(
        body,
        grid=(bsz // nb,),
        in_specs=[
            pl.BlockSpec((nb, half, 2 * d), lambda i: (i, 0, 0)),
        ],
        out_specs=[
            pl.BlockSpec((nb, 1, half), lambda i: (i, 0, 0)),
            pl.BlockSpec((nb, half, 16), lambda i: (i, 0, 0)),
        ],
        out_shape=[
            jax.ShapeDtypeStruct((bsz, 1, half), jnp.int32),
            jax.ShapeDtypeStruct((bsz, half, 16), jnp.float32),
        ],
    )(m2)

    idx3 = idx_all.reshape(bsz, _NSTREAM, half // _NSTREAM)
    zz = jnp.zeros((unm, _F), jnp.float32)

    nw = 32
    bpw = bsz // nw
    nch = c // _F
    mesh = plsc.VectorSubcoreMesh(core_axis_name="c", subcore_axis_name="s")
    sc_body = functools.partial(_sc_merge_body, half=half, unm=unm, c=c,
                                bpw=bpw, nch=nch)
    merge = pl.kernel(
        sc_body,
        mesh=mesh,
        out_type=jax.ShapeDtypeStruct((bsz, nout, c), jnp.float32),
        scratch_types=[
            pltpu.VMEM((half, _F), jnp.float32),
            pltpu.VMEM((_NSTREAM, half // _NSTREAM), jnp.int32),
            pltpu.VMEM((half, 16), jnp.float32),
            pltpu.SemaphoreType.DMA,
            pltpu.SemaphoreType.DMA,
            pltpu.SemaphoreType.DMA,
            pltpu.VMEM_SHARED((16 * nout, _F), jnp.float32),
        ],
    )
    return merge(x2, idx3, invw, zz)


# speedup vs baseline: 1.0853x; 1.0070x over previous
import functools

import jax
import jax.numpy as jnp
from jax import lax
from jax.experimental import pallas as pl
from jax.experimental.pallas import tpu as pltpu
from jax.experimental.pallas import tpu_sc as plsc

_R = 144


def _rownorm_sumsq(v):
    sq = v * v
    n = sq.shape[1]
    s = sq[:, 0:8]
    for k in range(1, n // 8):
        s = s + sq[:, 8 * k:8 * (k + 1)]
    w = 8
    while w > 1:
        s = s[:, :w // 2] + s[:, w // 2:w]
        w //= 2
    return s


def _select_body(m_ref, idx_ref, w_ref, *, half, r, big, nb):
    for bb in range(nb):
        _select_one(m_ref, idx_ref, w_ref, bb, half=half, r=r, big=big)


def _select_one(m_ref, idx_ref, w_ref, bb, *, half, r, big):
    mm = m_ref[bb]
    d = mm.shape[1] // 2
    a = mm[:, :d]
    b = mm[:, d:]
    a = a / jnp.sqrt(_rownorm_sumsq(a))
    b = b / jnp.sqrt(_rownorm_sumsq(b))
    s = lax.dot_general(a, b, (((1,), (1,)), ((), ())),
                        preferred_element_type=jnp.float32)

    nm = jnp.max(s, axis=1, keepdims=True)
    ii = lax.broadcasted_iota(jnp.int32, (half, half), 0)
    jj = lax.broadcasted_iota(jnp.int32, (half, half), 1)
    nidx = jnp.min(jnp.where(s == nm, jj, big), axis=1, keepdims=True)

    nm_row = jnp.swapaxes(nm, 0, 1)

    cmp = ((nm_row > nm) | ((nm_row == nm) & (jj < ii))).astype(jnp.float32)
    rank = lax.dot_general(cmp, jnp.ones((half, 1), jnp.float32),
                           (((1,), (0,)), ((), ())),
                           preferred_element_type=jnp.float32)
    rank = rank.astype(jnp.int32)

    unm = half - r
    o_idx = jnp.where(rank >= r, rank - r, unm + nidx)
    nout = unm + half
    b_global = pl.program_id(0) * 4 + bb
    o_idx = o_idx + (b_global // 8) * nout
    idx_ref[bb] = jnp.swapaxes(o_idx, 0, 1)

    dmat = ((nidx == jj) & (rank < r)).astype(jnp.float32)
    cnt = lax.dot_general(jnp.ones((1, half), jnp.float32), dmat,
                          (((1,), (0,)), ((), ())),
                          preferred_element_type=jnp.float32)
    invw = 1.0 / (1.0 + jnp.swapaxes(cnt, 0, 1))
    w_ref[bb] = jnp.broadcast_to(invw, (half, 16))


_F = 128
_NSTREAM = 3


def _sc_merge_body(x2_hbm, idx_hbm, invw_hbm, zz_hbm, out_hbm,
                   esrc, idxv, wv, sem0, sem1, sem2, acc_sh,
                   *, half, unm, c, bpw, nch):
    nc = 2
    wid = lax.axis_index("s") * nc + lax.axis_index("c")
    nout = unm + half
    a0 = lax.axis_index("s") * nout
    for bb in range(bpw):
        bidx = wid * bpw + bb
        pltpu.sync_copy(idx_hbm.at[bidx], idxv)
        pltpu.sync_copy(invw_hbm.at[bidx], wv)
        for ch in range(nch):
            c0 = ch * _F
            h0 = pltpu.async_copy(zz_hbm, acc_sh.at[pl.ds(a0, unm)], sem0)
            h1 = pltpu.async_copy(x2_hbm.at[bidx, :, pl.ds(c + c0, _F)],
                                  acc_sh.at[pl.ds(a0 + unm, half)], sem1)
            h2 = pltpu.async_copy(x2_hbm.at[bidx, :, pl.ds(c0, _F)],
                                  esrc, sem2)
            h0.wait()
            h1.wait()
            h2.wait()
            hs = []
            for j, sm in zip(range(_NSTREAM), (sem0, sem1, sem2)):
                hs.append(pltpu.async_copy(esrc.at[pl.ds(j * 96, 96)],
                                           acc_sh.at[idxv.at[j]], sm,
                                           add=True))
            for h in hs:
                h.wait()

            h0 = pltpu.async_copy(acc_sh.at[pl.ds(a0, unm)],
                                  out_hbm.at[bidx, pl.ds(0, unm),
                                             pl.ds(c0, _F)], sem0)
            pltpu.sync_copy(acc_sh.at[pl.ds(a0 + unm, half)], esrc)

            def mrow(rr, carry):
                w16 = wv[rr]
                for k in range(_F // 16):
                    sl = pl.ds(k * 16, 16)
                    esrc[rr, sl] = esrc[rr, sl] * w16
                return carry

            lax.fori_loop(0, half, mrow, 0)
            h0.wait()
            pltpu.sync_copy(esrc,
                            out_hbm.at[bidx, pl.ds(unm, half), pl.ds(c0, _F)])


def kernel(x, metric):
    bsz, t, c = x.shape
    d = metric.shape[-1]
    half = t // 2
    r = min(_R, half)
    unm = half - r
    nout = unm + half

    x2 = x.reshape(bsz, half, 2 * c)
    m2 = metric.reshape(bsz, half, 2 * d)

    nb = 4
    body = functools.partial(_select_body, half=half, r=r, big=1 << 30,
                             nb=nb)
    idx_all, invw = pl.pallas_call(
        body,
        grid=(bsz // nb,),
        in_specs=[
            pl.BlockSpec((nb, half, 2 * d), lambda i: (i, 0, 0)),
        ],
        out_specs=[
            pl.BlockSpec((nb, 1, half), lambda i: (i, 0, 0)),
            pl.BlockSpec((nb, half, 16), lambda i: (i, 0, 0)),
        ],
        out_shape=[
            jax.ShapeDtypeStruct((bsz, 1, half), jnp.int32),
            jax.ShapeDtypeStruct((bsz, half, 16), jnp.float32),
        ],
    )(m2)

    idx3 = idx_all.reshape(bsz, _NSTREAM, half // _NSTREAM)
    zz = jnp.zeros((unm, _F), jnp.float32)

    nw = 32
    bpw = bsz // nw
    nch = c // _F
    mesh = plsc.VectorSubcoreMesh(core_axis_name="c", subcore_axis_name="s")
    sc_body = functools.partial(_sc_merge_body, half=half, unm=unm, c=c,
                                bpw=bpw, nch=nch)
    merge = pl.kernel(
        sc_body,
        mesh=mesh,
        out_type=jax.ShapeDtypeStruct((bsz, nout, c), jnp.float32),
        scratch_types=[
            pltpu.VMEM((half, _F), jnp.float32),
            pltpu.VMEM((_NSTREAM, half // _NSTREAM), jnp.int32),
            pltpu.VMEM((half, 16), jnp.float32),
            pltpu.SemaphoreType.DMA,
            pltpu.SemaphoreType.DMA,
            pltpu.SemaphoreType.DMA,
            pltpu.VMEM_SHARED((16 * nout, _F), jnp.float32),
        ],
    )
    return merge(x2, idx3, invw, zz)
